# baseline (speedup 1.0000x reference)
import functools
import jax
import jax.numpy as jnp
from jax.experimental import pallas as pl
from jax.experimental.pallas import tpu as pltpu

_Q, _K, _D, _OUT = 16, 100000, 128, 18
_MIN_DIST = float(_OUT)
_BLK = 2000
_NBLK = _K // _BLK


def _tc_body(x_ref, obs_ref, act_ref, out_ref, minv_ref, mini_ref, vout_ref, sem):
    b = pl.program_id(0)

    @pl.when(b == 0)
    def _init():
        for q in range(_Q):
            minv_ref[q] = jnp.inf
            mini_ref[q] = 0

    x = x_ref[...]
    deltas = jnp.concatenate([x[:1], x[:-1] - x[1:]], axis=0)
    dabs = jnp.abs(deltas)

    obs = obs_ref[...]
    oabs = jnp.abs(obs)
    base = b * _BLK
    rows = jax.lax.broadcasted_iota(jnp.int32, (_BLK,), 0) + base

    for q in range(_Q):
        num = jnp.abs(obs - deltas[q][None, :])
        den = oabs + dabs[q][None, :]
        term = jnp.where(den > 0, num / jnp.where(den > 0, den, 1.0), 0.0)
        dist = jnp.sum(term, axis=1)
        bmin = jnp.min(dist)
        bidx = jnp.min(jnp.where(dist == bmin, rows, jnp.int32(2**31 - 1)))

        @pl.when(bmin < minv_ref[q])
        def _upd():
            minv_ref[q] = bmin
            mini_ref[q] = bidx

    @pl.when(b == _NBLK - 1)
    def _fin():
        for q in range(_Q):
            copy = pltpu.make_async_copy(
                act_ref.at[pl.ds(mini_ref[q], 1)], vout_ref.at[pl.ds(q, 1)], sem
            )
            copy.start()
            copy.wait()
        for q in range(_Q):
            row = vout_ref[pl.ds(q, 1), :]
            out_ref[pl.ds(q, 1), :] = jnp.where(
                minv_ref[q] < _MIN_DIST, row, jnp.zeros_like(row)
            )


@jax.jit
def kernel(x, observations, actions):
    grid_spec = pltpu.PrefetchScalarGridSpec(
        num_scalar_prefetch=0,
        grid=(_NBLK,),
        in_specs=[
            pl.BlockSpec((_Q, _D), lambda b: (0, 0)),
            pl.BlockSpec((_BLK, _D), lambda b: (b, 0)),
            pl.BlockSpec(memory_space=pl.ANY),
        ],
        out_specs=pl.BlockSpec((_Q, _OUT), lambda b: (0, 0)),
        scratch_shapes=[
            pltpu.SMEM((_Q,), jnp.float32),
            pltpu.SMEM((_Q,), jnp.int32),
            pltpu.VMEM((_Q, _OUT), jnp.float32),
            pltpu.SemaphoreType.DMA,
        ],
    )
    return pl.pallas_call(
        _tc_body,
        grid_spec=grid_spec,
        out_shape=jax.ShapeDtypeStruct((_Q, _OUT), jnp.float32),
    )(x, observations, actions)


# speedup vs baseline: 1.4468x; 1.4468x over previous
import functools
import jax
import jax.numpy as jnp
from jax.experimental import pallas as pl
from jax.experimental.pallas import tpu as pltpu

_Q, _K, _D, _OUT = 16, 100000, 128, 18
_MIN_DIST = float(_OUT)
_BLK = 2000
_NBLK = _K // _BLK


def _tc_body(x_ref, obs_ref, act_ref, out_ref, minv_ref, mini_ref, vout_ref, sem):
    b = pl.program_id(0)

    @pl.when(b == 0)
    def _init():
        for q in range(_Q):
            minv_ref[q] = jnp.inf
            mini_ref[q] = 0

    x = x_ref[...]
    deltas = jnp.concatenate([x[:1], x[:-1] - x[1:]], axis=0)
    dabs = jnp.abs(deltas)

    obs = obs_ref[...]
    oabs = jnp.abs(obs)
    base = b * _BLK
    ones = jnp.ones((_D, 8), jnp.float32)

    for q in range(_Q):
        num = jnp.abs(obs - deltas[q][None, :])
        den = jnp.maximum(oabs + dabs[q][None, :], 1e-30)
        term = num / den
        dist8 = jax.lax.dot_general(
            term, ones, (((1,), (0,)), ((), ())),
            preferred_element_type=jnp.float32,
        )
        bmin = jnp.min(dist8)

        @pl.when(bmin < minv_ref[q])
        def _upd():
            rows = jax.lax.broadcasted_iota(jnp.int32, (_BLK, 8), 0) + base
            bidx = jnp.min(
                jnp.where(dist8 == bmin, rows, jnp.int32(2**31 - 1))
            )
            minv_ref[q] = bmin
            mini_ref[q] = bidx

    @pl.when(b == _NBLK - 1)
    def _fin():
        for q in range(_Q):
            copy = pltpu.make_async_copy(
                act_ref.at[pl.ds(mini_ref[q], 1)], vout_ref.at[pl.ds(q, 1)], sem
            )
            copy.start()
            copy.wait()
        for q in range(_Q):
            row = vout_ref[pl.ds(q, 1), :]
            out_ref[pl.ds(q, 1), :] = jnp.where(
                minv_ref[q] < _MIN_DIST, row, jnp.zeros_like(row)
            )


@jax.jit
def kernel(x, observations, actions):
    grid_spec = pltpu.PrefetchScalarGridSpec(
        num_scalar_prefetch=0,
        grid=(_NBLK,),
        in_specs=[
            pl.BlockSpec((_Q, _D), lambda b: (0, 0)),
            pl.BlockSpec((_BLK, _D), lambda b: (b, 0)),
            pl.BlockSpec(memory_space=pl.ANY),
        ],
        out_specs=pl.BlockSpec((_Q, _OUT), lambda b: (0, 0)),
        scratch_shapes=[
            pltpu.SMEM((_Q,), jnp.float32),
            pltpu.SMEM((_Q,), jnp.int32),
            pltpu.VMEM((_Q, _OUT), jnp.float32),
            pltpu.SemaphoreType.DMA,
        ],
    )
    return pl.pallas_call(
        _tc_body,
        grid_spec=grid_spec,
        out_shape=jax.ShapeDtypeStruct((_Q, _OUT), jnp.float32),
    )(x, observations, actions)
